# Initial kernel scaffold; baseline (speedup 1.0000x reference)
import jax
import jax.numpy as jnp
from jax.experimental import pallas as pl


def kernel(seq1, seq2, edge_index, W1, b1, W2, b2, W3, b3, alpha, Wd, bd):
    raise NotImplementedError("write your pallas kernel here")


# speedup vs baseline: 14.7424x; 14.7424x over previous
import functools

import jax
import jax.numpy as jnp
from jax import lax
from jax.experimental import pallas as pl
from jax.experimental.pallas import tpu as pltpu
from jax.experimental.pallas import tpu_sc as plsc

N = 10000
NPAD = 10240
E = 320000
F = 128
NC = 2
NS = 16
NW = NC * NS

WIN = 128
CHW = 16
NCH = 10
EPT = NCH * CHW * WIN
EPAD = NS * EPT

DEG_WPW = 80
DPT = NPAD // NS

_mesh = plsc.VectorSubcoreMesh(
    core_axis_name="c", subcore_axis_name="s", num_cores=NC, num_subcores=NS)


@functools.partial(
    pl.kernel,
    out_type=jax.ShapeDtypeStruct((NC, NPAD), jnp.float32),
    mesh=_mesh,
    scratch_types=[
        pltpu.VMEM((DEG_WPW, WIN), jnp.int32),
        pltpu.VMEM((DPT,), jnp.float32),
        pltpu.VMEM((WIN,), jnp.float32),
        pltpu.VMEM_SHARED((NPAD,), jnp.float32),
    ],
)
def _deg_kernel(dst_hbm, deg_out, dst_v, zbuf, ones_v, deg_sh):
    c = lax.axis_index("c")
    s = lax.axis_index("s")
    wid = s * NC + c

    def zfill(i, carry):
        zbuf[pl.ds(i * 16, 16)] = jnp.zeros((16,), jnp.float32)
        return carry
    lax.fori_loop(0, DPT // 16, zfill, 0)

    def ofill(i, carry):
        ones_v[pl.ds(i * 16, 16)] = jnp.ones((16,), jnp.float32)
        return carry
    lax.fori_loop(0, WIN // 16, ofill, 0)

    pltpu.sync_copy(zbuf, deg_sh.at[pl.ds(s * DPT, DPT)])
    pltpu.sync_copy(dst_hbm.at[wid], dst_v)
    plsc.subcore_barrier()

    def w(j, carry):
        pltpu.sync_copy(ones_v, deg_sh.at[dst_v.at[j]], add=True)
        return carry
    lax.fori_loop(0, DEG_WPW, w, 0)

    plsc.subcore_barrier()
    pltpu.sync_copy(deg_sh.at[pl.ds(s * DPT, DPT)],
                    deg_out.at[c, pl.ds(s * DPT, DPT)])


@functools.partial(
    pl.kernel,
    out_type=jax.ShapeDtypeStruct((NC, NPAD, F), jnp.float32),
    mesh=_mesh,
    scratch_types=[
        pltpu.VMEM((CHW, WIN), jnp.int32),
        pltpu.VMEM((CHW, WIN), jnp.int32),
        pltpu.VMEM((WIN, F), jnp.float32),
        pltpu.VMEM_SHARED((NPAD, F), jnp.float32),
        pltpu.SemaphoreType.DMA,
    ],
)
def _layer_kernel(g_hbm, src_hbm, dst_hbm, out_hbm, src_v, dst_v, buf, agg_sh, sem):
    c = lax.axis_index("c")
    s = lax.axis_index("s")

    pltpu.sync_copy(g_hbm.at[pl.ds(c * NPAD + s * DPT, DPT)],
                    agg_sh.at[pl.ds(s * DPT, DPT)])
    plsc.subcore_barrier()

    def chunk(ch, carry):
        pltpu.sync_copy(src_hbm.at[c, s, ch], src_v)
        pltpu.sync_copy(dst_hbm.at[s, ch], dst_v)

        def w(j, carry2):
            pltpu.async_copy(g_hbm.at[src_v.at[j]], buf, sem).wait()
            pltpu.sync_copy(buf, agg_sh.at[dst_v.at[j]], add=True)
            return carry2
        lax.fori_loop(0, CHW, w, 0)
        return carry
    lax.fori_loop(0, NCH, chunk, 0)

    plsc.subcore_barrier()
    pltpu.sync_copy(agg_sh.at[pl.ds(s * DPT, DPT)],
                    out_hbm.at[c, pl.ds(s * DPT, DPT)])


RB = 2048
_GRID = (2, NPAD // RB)

_DOT = dict(preferred_element_type=jnp.float32, precision=lax.Precision.HIGHEST)


def _dis_of(degp_ref):
    d = degp_ref[0, :, 0] + degp_ref[1, :, 0] + 1.0
    return lax.rsqrt(jnp.maximum(d, 1.0))


def _entry_body(seq_ref, degp_ref, w_ref, b_ref, g_ref):
    dis = _dis_of(degp_ref)
    h = jnp.dot(seq_ref[0], w_ref[...], **_DOT) + b_ref[0][None, :]
    g_ref[0] = h * dis[:, None]


def _mid_body(s_ref, degp_ref, w_ref, b_ref, alpha_ref, g_ref):
    dis = _dis_of(degp_ref)
    a = s_ref[0] * dis[:, None]
    x = jnp.where(a >= 0.0, a, alpha_ref[0, 0] * a)
    h = jnp.dot(x, w_ref[...], **_DOT) + b_ref[0][None, :]
    g_ref[0] = h * dis[:, None]


def _final_body(s_ref, degp_ref, wd_ref, alpha_ref, bd_ref, out_ref):
    d = degp_ref[0, :, 0] + degp_ref[1, :, 0] + 1.0
    dis = lax.rsqrt(jnp.maximum(d, 1.0))
    alpha = alpha_ref[0, 0]
    a1 = s_ref[0] * dis[:, None]
    h1 = jnp.where(a1 >= 0.0, a1, alpha * a1)
    a2 = s_ref[1] * dis[:, None]
    h2 = jnp.where(a2 >= 0.0, a2, alpha * a2)
    row = lax.broadcasted_iota(jnp.int32, (NPAD, F), 0)
    h1m = jnp.where(row < N, h1, 0.0)
    cvec = jax.nn.sigmoid(jnp.sum(h1m, axis=0, keepdims=True) * (1.0 / N))
    cw = lax.dot_general(wd_ref[...], cvec, (((1,), (1,)), ((), ())), **_DOT)
    bd = bd_ref[0, 0]
    out_ref[0] = jnp.dot(h1, cw, **_DOT) + bd
    out_ref[1] = jnp.dot(h2, cw, **_DOT) + bd


_entry = pl.pallas_call(
    _entry_body,
    grid=_GRID,
    in_specs=[
        pl.BlockSpec((1, RB, F), lambda i, j: (i, j, 0)),
        pl.BlockSpec((2, RB, 1), lambda i, j: (0, j, 0)),
        pl.BlockSpec((F, F), lambda i, j: (0, 0)),
        pl.BlockSpec((1, F), lambda i, j: (0, 0)),
    ],
    out_specs=pl.BlockSpec((1, RB, F), lambda i, j: (i, j, 0)),
    out_shape=jax.ShapeDtypeStruct((2, NPAD, F), jnp.float32),
)

_mid = pl.pallas_call(
    _mid_body,
    grid=_GRID,
    in_specs=[
        pl.BlockSpec((1, RB, F), lambda i, j: (i, j, 0)),
        pl.BlockSpec((2, RB, 1), lambda i, j: (0, j, 0)),
        pl.BlockSpec((F, F), lambda i, j: (0, 0)),
        pl.BlockSpec((1, F), lambda i, j: (0, 0)),
        pl.BlockSpec((1, 1), lambda i, j: (0, 0)),
    ],
    out_specs=pl.BlockSpec((1, RB, F), lambda i, j: (i, j, 0)),
    out_shape=jax.ShapeDtypeStruct((2, NPAD, F), jnp.float32),
)

_final = pl.pallas_call(
    _final_body,
    out_shape=jax.ShapeDtypeStruct((2, NPAD, 1), jnp.float32),
    compiler_params=pltpu.CompilerParams(vmem_limit_bytes=100 * 1024 * 1024),
)


def kernel(seq1, seq2, edge_index, W1, b1, W2, b2, W3, b3, alpha, Wd, bd):
    src = edge_index[0].astype(jnp.int32)
    dst = edge_index[1].astype(jnp.int32)

    P = EPAD - E
    pad_src = jnp.arange(P, dtype=jnp.int32) % N
    pad_dst = N + jnp.arange(P, dtype=jnp.int32) % (NPAD - N)
    srcp = jnp.concatenate([src, pad_src])
    dstp = jnp.concatenate([dst, pad_dst])

    degp = _deg_kernel(dstp.reshape(NW, DEG_WPW, WIN))
    degp = degp.reshape(2, NPAD, 1)

    src4 = srcp.reshape(NS, NCH, CHW, WIN)
    src_sc = jnp.stack([src4, src4 + NPAD])
    dst4 = dstp.reshape(NS, NCH, CHW, WIN)

    pad = ((0, 0), (0, NPAD - N), (0, 0))
    seqs = jnp.pad(jnp.stack([seq1, seq2]), pad)
    alpha_a = alpha.reshape(1, 1)
    bd_a = bd.reshape(1, 1)

    g = _entry(seqs, degp, W1, b1.reshape(1, F))
    s_agg = _layer_kernel(g.reshape(2 * NPAD, F), src_sc, dst4)
    g = _mid(s_agg, degp, W2, b2.reshape(1, F), alpha_a)
    s_agg = _layer_kernel(g.reshape(2 * NPAD, F), src_sc, dst4)
    g = _mid(s_agg, degp, W3, b3.reshape(1, F), alpha_a)
    s_agg = _layer_kernel(g.reshape(2 * NPAD, F), src_sc, dst4)
    out2 = _final(s_agg, degp, Wd, alpha_a, bd_a)
    return jnp.concatenate([out2[0, :N, 0], out2[1, :N, 0]])


# speedup vs baseline: 22.9327x; 1.5556x over previous
import functools

import jax
import jax.numpy as jnp
from jax import lax
from jax.experimental import pallas as pl
from jax.experimental.pallas import tpu as pltpu
from jax.experimental.pallas import tpu_sc as plsc

N = 10000
NPAD = 10240
E = 320000
F = 128
NC = 2
NS = 16
NW = NC * NS

WIN = 128
CHW = 40
NCH = 4
EPT = NCH * CHW * WIN
EPAD = NS * EPT

DEG_WPW = 80
DPT = NPAD // NS

_mesh = plsc.VectorSubcoreMesh(
    core_axis_name="c", subcore_axis_name="s", num_cores=NC, num_subcores=NS)


@functools.partial(
    pl.kernel,
    out_type=jax.ShapeDtypeStruct((NC, NPAD), jnp.float32),
    mesh=_mesh,
    scratch_types=[
        pltpu.VMEM((DEG_WPW, WIN), jnp.int32),
        pltpu.VMEM((DPT,), jnp.float32),
        pltpu.VMEM((WIN,), jnp.float32),
        pltpu.VMEM_SHARED((NPAD,), jnp.float32),
    ],
)
def _deg_kernel(dst_hbm, deg_out, dst_v, zbuf, ones_v, deg_sh):
    c = lax.axis_index("c")
    s = lax.axis_index("s")
    wid = s * NC + c

    def zfill(i, carry):
        zbuf[pl.ds(i * 16, 16)] = jnp.zeros((16,), jnp.float32)
        return carry
    lax.fori_loop(0, DPT // 16, zfill, 0)

    def ofill(i, carry):
        ones_v[pl.ds(i * 16, 16)] = jnp.ones((16,), jnp.float32)
        return carry
    lax.fori_loop(0, WIN // 16, ofill, 0)

    pltpu.sync_copy(zbuf, deg_sh.at[pl.ds(s * DPT, DPT)])
    pltpu.sync_copy(dst_hbm.at[wid], dst_v)
    plsc.subcore_barrier()

    def w(j, carry):
        pltpu.sync_copy(ones_v, deg_sh.at[dst_v.at[j]], add=True)
        return carry
    lax.fori_loop(0, DEG_WPW, w, 0)

    plsc.subcore_barrier()
    pltpu.sync_copy(deg_sh.at[pl.ds(s * DPT, DPT)],
                    deg_out.at[c, pl.ds(s * DPT, DPT)])


@functools.partial(
    pl.kernel,
    out_type=jax.ShapeDtypeStruct((NC, NPAD, F), jnp.float32),
    mesh=_mesh,
    scratch_types=[
        pltpu.VMEM((CHW, WIN), jnp.int32),
        pltpu.VMEM((CHW, WIN), jnp.int32),
        pltpu.VMEM((WIN, F), jnp.float32),
        pltpu.VMEM((WIN, F), jnp.float32),
        pltpu.VMEM_SHARED((NPAD, F), jnp.float32),
        pltpu.SemaphoreType.DMA,
        pltpu.SemaphoreType.DMA,
    ],
)
def _layer_kernel(g_hbm, src_hbm, dst_hbm, out_hbm, src_v, dst_v, buf_a, buf_b,
                  agg_sh, sem_a, sem_b):
    c = lax.axis_index("c")
    s = lax.axis_index("s")

    pltpu.sync_copy(g_hbm.at[pl.ds(c * NPAD + s * DPT, DPT)],
                    agg_sh.at[pl.ds(s * DPT, DPT)])
    plsc.subcore_barrier()

    def chunk(ch, carry):
        pltpu.sync_copy(src_hbm.at[c, s, ch], src_v)
        pltpu.sync_copy(dst_hbm.at[s, ch], dst_v)
        pltpu.async_copy(g_hbm.at[src_v.at[0]], buf_a, sem_a)
        pltpu.async_copy(g_hbm.at[src_v.at[1]], buf_b, sem_b)

        def pair(t, carry2):
            w0 = 2 * t
            pltpu.make_async_copy(g_hbm.at[src_v.at[w0]], buf_a, sem_a).wait()
            pltpu.sync_copy(buf_a, agg_sh.at[dst_v.at[w0]], add=True)

            @pl.when(w0 + 2 < CHW)
            def _():
                pltpu.async_copy(g_hbm.at[src_v.at[w0 + 2]], buf_a, sem_a)

            pltpu.make_async_copy(g_hbm.at[src_v.at[w0 + 1]], buf_b, sem_b).wait()
            pltpu.sync_copy(buf_b, agg_sh.at[dst_v.at[w0 + 1]], add=True)

            @pl.when(w0 + 3 < CHW)
            def _():
                pltpu.async_copy(g_hbm.at[src_v.at[w0 + 3]], buf_b, sem_b)
            return carry2
        lax.fori_loop(0, CHW // 2, pair, 0)
        return carry
    lax.fori_loop(0, NCH, chunk, 0)

    plsc.subcore_barrier()
    pltpu.sync_copy(agg_sh.at[pl.ds(s * DPT, DPT)],
                    out_hbm.at[c, pl.ds(s * DPT, DPT)])


RB = 2048
_GRID = (2, NPAD // RB)

_DOT = dict(preferred_element_type=jnp.float32, precision=lax.Precision.HIGHEST)


def _dis_of(degp_ref):
    d = degp_ref[0, :, 0] + degp_ref[1, :, 0] + 1.0
    return lax.rsqrt(jnp.maximum(d, 1.0))


def _entry_body(seq_ref, degp_ref, w_ref, b_ref, g_ref):
    dis = _dis_of(degp_ref)
    h = jnp.dot(seq_ref[0], w_ref[...], **_DOT) + b_ref[0][None, :]
    g_ref[0] = h * dis[:, None]


def _mid_body(s_ref, degp_ref, w_ref, b_ref, alpha_ref, g_ref):
    dis = _dis_of(degp_ref)
    a = s_ref[0] * dis[:, None]
    x = jnp.where(a >= 0.0, a, alpha_ref[0, 0] * a)
    h = jnp.dot(x, w_ref[...], **_DOT) + b_ref[0][None, :]
    g_ref[0] = h * dis[:, None]


def _final_body(s_ref, degp_ref, wd_ref, alpha_ref, bd_ref, out_ref):
    d = degp_ref[0, :, 0] + degp_ref[1, :, 0] + 1.0
    dis = lax.rsqrt(jnp.maximum(d, 1.0))
    alpha = alpha_ref[0, 0]
    a1 = s_ref[0] * dis[:, None]
    h1 = jnp.where(a1 >= 0.0, a1, alpha * a1)
    a2 = s_ref[1] * dis[:, None]
    h2 = jnp.where(a2 >= 0.0, a2, alpha * a2)
    row = lax.broadcasted_iota(jnp.int32, (NPAD, F), 0)
    h1m = jnp.where(row < N, h1, 0.0)
    cvec = jax.nn.sigmoid(jnp.sum(h1m, axis=0, keepdims=True) * (1.0 / N))
    cw = lax.dot_general(wd_ref[...], cvec, (((1,), (1,)), ((), ())), **_DOT)
    bd = bd_ref[0, 0]
    out_ref[0] = jnp.dot(h1, cw, **_DOT) + bd
    out_ref[1] = jnp.dot(h2, cw, **_DOT) + bd


_entry = pl.pallas_call(
    _entry_body,
    grid=_GRID,
    in_specs=[
        pl.BlockSpec((1, RB, F), lambda i, j: (i, j, 0)),
        pl.BlockSpec((2, RB, 1), lambda i, j: (0, j, 0)),
        pl.BlockSpec((F, F), lambda i, j: (0, 0)),
        pl.BlockSpec((1, F), lambda i, j: (0, 0)),
    ],
    out_specs=pl.BlockSpec((1, RB, F), lambda i, j: (i, j, 0)),
    out_shape=jax.ShapeDtypeStruct((2, NPAD, F), jnp.float32),
)

_mid = pl.pallas_call(
    _mid_body,
    grid=_GRID,
    in_specs=[
        pl.BlockSpec((1, RB, F), lambda i, j: (i, j, 0)),
        pl.BlockSpec((2, RB, 1), lambda i, j: (0, j, 0)),
        pl.BlockSpec((F, F), lambda i, j: (0, 0)),
        pl.BlockSpec((1, F), lambda i, j: (0, 0)),
        pl.BlockSpec((1, 1), lambda i, j: (0, 0)),
    ],
    out_specs=pl.BlockSpec((1, RB, F), lambda i, j: (i, j, 0)),
    out_shape=jax.ShapeDtypeStruct((2, NPAD, F), jnp.float32),
)

_final = pl.pallas_call(
    _final_body,
    out_shape=jax.ShapeDtypeStruct((2, NPAD, 1), jnp.float32),
    compiler_params=pltpu.CompilerParams(vmem_limit_bytes=100 * 1024 * 1024),
)


def kernel(seq1, seq2, edge_index, W1, b1, W2, b2, W3, b3, alpha, Wd, bd):
    src = edge_index[0].astype(jnp.int32)
    dst = edge_index[1].astype(jnp.int32)

    P = EPAD - E
    pad_src = jnp.arange(P, dtype=jnp.int32) % N
    pad_dst = N + jnp.arange(P, dtype=jnp.int32) % (NPAD - N)
    srcp = jnp.concatenate([src, pad_src])
    dstp = jnp.concatenate([dst, pad_dst])

    degp = _deg_kernel(dstp.reshape(NW, DEG_WPW, WIN))
    degp = degp.reshape(2, NPAD, 1)

    src4 = srcp.reshape(NS, NCH, CHW, WIN)
    src_sc = jnp.stack([src4, src4 + NPAD])
    dst4 = dstp.reshape(NS, NCH, CHW, WIN)

    pad = ((0, 0), (0, NPAD - N), (0, 0))
    seqs = jnp.pad(jnp.stack([seq1, seq2]), pad)
    alpha_a = alpha.reshape(1, 1)
    bd_a = bd.reshape(1, 1)

    g = _entry(seqs, degp, W1, b1.reshape(1, F))
    s_agg = _layer_kernel(g.reshape(2 * NPAD, F), src_sc, dst4)
    g = _mid(s_agg, degp, W2, b2.reshape(1, F), alpha_a)
    s_agg = _layer_kernel(g.reshape(2 * NPAD, F), src_sc, dst4)
    g = _mid(s_agg, degp, W3, b3.reshape(1, F), alpha_a)
    s_agg = _layer_kernel(g.reshape(2 * NPAD, F), src_sc, dst4)
    out2 = _final(s_agg, degp, Wd, alpha_a, bd_a)
    return jnp.concatenate([out2[0, :N, 0], out2[1, :N, 0]])
